# Initial kernel scaffold; baseline (speedup 1.0000x reference)
import jax
import jax.numpy as jnp
from jax.experimental import pallas as pl


def kernel(x, edge_index, edge_weight, batch, Wq1, bq1, Wk1, bk1, Wv1, bv1, We1, Ws1, bs1, Wq2, bq2, Wk2, bk2, Wv2, bv2, We2, Ws2, bs2, Wl, bl):
    raise NotImplementedError("write your pallas kernel here")


# speedup vs baseline: 4.0781x; 4.0781x over previous
import functools

import jax
import jax.numpy as jnp
from jax import lax
from jax.experimental import pallas as pl
from jax.experimental.pallas import tpu as pltpu
from jax.experimental.pallas import tpu_sc as plsc

N = 10000
E = 320000
H = 128
B = 64
OUT = 10
NP = 10112
NBLOCKS = NP // 128

NC = 2
NS = 16
LN = 16
NTILES = NC * NS
EPT = E // NTILES
G = 80
NBLK = EPT // G
RPT = 632
NS_ROWS = 128


def _lift_kernel(x_ref, w_ref, b_ref, q_ref, k_ref, v_ref, s_ref):
    y = jnp.dot(x_ref[...], w_ref[...], preferred_element_type=jnp.float32)
    y = y + b_ref[...]
    q_ref[...] = y[:, :256]
    k_ref[...] = y[:, 256:384]
    v_ref[...] = y[:, 384:512]
    s_ref[...] = y[:, 512:]


def _lift(xp, W, b):
    return pl.pallas_call(
        _lift_kernel,
        grid=(NBLOCKS,),
        in_specs=[
            pl.BlockSpec((128, H), lambda i: (i, 0)),
            pl.BlockSpec((H, 640), lambda i: (0, 0)),
            pl.BlockSpec((1, 640), lambda i: (0, 0)),
        ],
        out_specs=[
            pl.BlockSpec((128, 256), lambda i: (i, 0)),
            pl.BlockSpec((128, H), lambda i: (i, 0)),
            pl.BlockSpec((128, H), lambda i: (i, 0)),
            pl.BlockSpec((128, H), lambda i: (i, 0)),
        ],
        out_shape=[
            jax.ShapeDtypeStruct((NP, 256), jnp.float32),
            jax.ShapeDtypeStruct((NP, H), jnp.float32),
            jax.ShapeDtypeStruct((NP, H), jnp.float32),
            jax.ShapeDtypeStruct((NP, H), jnp.float32),
        ],
    )(xp, W, b)


def _node_out(av, sv, s, i):
    ohi = jnp.where(
        lax.broadcasted_iota(jnp.int32, (128, 128), 0) == i, 1.0, 0.0)
    den = jnp.dot(sv, ohi, preferred_element_type=jnp.float32)
    return av / (den + 1e-16) + s


def _mid_kernel(a0_ref, a1_ref, s0_ref, s1_ref, sk_ref, w_ref, b_ref,
                q_ref, k_ref, v_ref, s2_ref):
    i = pl.program_id(0)
    sv = s0_ref[...] + s1_ref[...]
    h = jnp.maximum(_node_out(a0_ref[...] + a1_ref[...], sv, sk_ref[...], i),
                    0.0)
    rows = i * 128 + lax.broadcasted_iota(jnp.int32, (128, 1), 0)
    h = jnp.where(rows < N, h, 0.0)
    y = jnp.dot(h, w_ref[...], preferred_element_type=jnp.float32)
    y = y + b_ref[...]
    q_ref[...] = y[:, :256]
    k_ref[...] = y[:, 256:384]
    v_ref[...] = y[:, 384:512]
    s2_ref[...] = y[:, 512:]


def _mid(a0, a1, s0, s1, sk, W, b):
    return pl.pallas_call(
        _mid_kernel,
        grid=(NBLOCKS,),
        in_specs=[
            pl.BlockSpec((128, H), lambda i: (i, 0)),
            pl.BlockSpec((128, H), lambda i: (i, 0)),
            pl.BlockSpec((128, 128), lambda i: (0, 0)),
            pl.BlockSpec((128, 128), lambda i: (0, 0)),
            pl.BlockSpec((128, H), lambda i: (i, 0)),
            pl.BlockSpec((H, 640), lambda i: (0, 0)),
            pl.BlockSpec((1, 640), lambda i: (0, 0)),
        ],
        out_specs=[
            pl.BlockSpec((128, 256), lambda i: (i, 0)),
            pl.BlockSpec((128, H), lambda i: (i, 0)),
            pl.BlockSpec((128, H), lambda i: (i, 0)),
            pl.BlockSpec((128, H), lambda i: (i, 0)),
        ],
        out_shape=[
            jax.ShapeDtypeStruct((NP, 256), jnp.float32),
            jax.ShapeDtypeStruct((NP, H), jnp.float32),
            jax.ShapeDtypeStruct((NP, H), jnp.float32),
            jax.ShapeDtypeStruct((NP, H), jnp.float32),
        ],
    )(a0, a1, s0, s1, sk, W, b)


def _fin_kernel(a0_ref, a1_ref, s0_ref, s1_ref, sk_ref, batch_ref,
                wlt_ref, bl_ref, out_ref, pooled_ref, cnt_ref):
    i = pl.program_id(0)

    @pl.when(i == 0)
    def _():
        pooled_ref[...] = jnp.zeros_like(pooled_ref)
        cnt_ref[...] = jnp.zeros_like(cnt_ref)

    sv = s0_ref[...] + s1_ref[...]
    h = jnp.maximum(_node_out(a0_ref[...] + a1_ref[...], sv, sk_ref[...], i),
                    0.0)
    rows = i * 128 + lax.broadcasted_iota(jnp.int32, (128, 1), 0)
    h = jnp.where(rows < N, h, 0.0)
    bvec = batch_ref[0, 0, :]
    seg = lax.broadcasted_iota(jnp.int32, (B, 128), 0)
    oh = jnp.where(seg == bvec[None, :], 1.0, 0.0)
    pooled_ref[...] += jnp.dot(oh, h, preferred_element_type=jnp.float32)
    cnt_ref[...] += jnp.dot(oh, jnp.ones((128, 128), jnp.float32),
                            preferred_element_type=jnp.float32)

    @pl.when(i == NBLOCKS - 1)
    def _():
        pooled = pooled_ref[...] / jnp.maximum(cnt_ref[...], 1.0)
        out_ref[...] = jnp.dot(pooled, wlt_ref[...],
                               preferred_element_type=jnp.float32) + bl_ref[...]


def _fin(a0, a1, s0, s1, sk, batch3d, wlt, bl):
    return pl.pallas_call(
        _fin_kernel,
        grid=(NBLOCKS,),
        in_specs=[
            pl.BlockSpec((128, H), lambda i: (i, 0)),
            pl.BlockSpec((128, H), lambda i: (i, 0)),
            pl.BlockSpec((128, 128), lambda i: (0, 0)),
            pl.BlockSpec((128, 128), lambda i: (0, 0)),
            pl.BlockSpec((128, H), lambda i: (i, 0)),
            pl.BlockSpec((1, 1, 128), lambda i: (i, 0, 0)),
            pl.BlockSpec((H, H), lambda i: (0, 0)),
            pl.BlockSpec((1, H), lambda i: (0, 0)),
        ],
        out_specs=pl.BlockSpec((B, H), lambda i: (0, 0)),
        out_shape=jax.ShapeDtypeStruct((B, H), jnp.float32),
        scratch_shapes=[
            pltpu.VMEM((B, H), jnp.float32),
            pltpu.VMEM((B, H), jnp.float32),
        ],
    )(a0, a1, s0, s1, sk, batch3d, wlt, bl)


_EDGE_KW = dict(
    out_type=(
        jax.ShapeDtypeStruct((NC, NP, H), jnp.float32),
        jax.ShapeDtypeStruct((NC, NS_ROWS, 128), jnp.float32),
    ),
    mesh=plsc.VectorSubcoreMesh(core_axis_name="c", subcore_axis_name="s"),
    compiler_params=pltpu.CompilerParams(needs_layout_passes=False),
    scratch_types=[
        pltpu.VMEM((G,), jnp.int32),
        pltpu.VMEM((G,), jnp.int32),
        pltpu.VMEM((G,), jnp.float32),
        pltpu.VMEM((H,), jnp.float32),
        pltpu.VMEM((G, 256), jnp.float32),
        pltpu.VMEM((G, H), jnp.float32),
        pltpu.VMEM((G, H), jnp.float32),
        pltpu.VMEM_SHARED((N, H), jnp.float32),
        pltpu.VMEM_SHARED((NS_ROWS, 128), jnp.float32),
        pltpu.SemaphoreType.DMA,
        pltpu.SemaphoreType.DMA,
        pltpu.SemaphoreType.DMA,
    ],
)


def _edge_body(qe_hbm, k_hbm, v_hbm, src_hbm, dst_hbm, ew_hbm, wec_hbm,
               zero_hbm, outv_hbm, outs_hbm,
               srcb, dstb, ewb, wecb, qrows, krows, vrows, accv, accs,
               sem1, sem2, sem3):
    cid = lax.axis_index("c")
    sid = lax.axis_index("s")
    lanes = jnp.arange(LN, dtype=jnp.int32)
    zvec = jnp.zeros((LN,), jnp.float32)
    pltpu.sync_copy(wec_hbm, wecb)

    @pl.when(sid < NS - 1)
    def _():
        pltpu.sync_copy(zero_hbm, accv.at[pl.ds(sid * RPT, RPT)])

    @pl.when(sid == NS - 1)
    def _():
        pltpu.sync_copy(zero_hbm.at[pl.ds(0, N - (NS - 1) * RPT)],
                        accv.at[pl.ds((NS - 1) * RPT, N - (NS - 1) * RPT)])

    @pl.when(sid < NS_ROWS // 16)
    def _():
        pltpu.sync_copy(zero_hbm.at[pl.ds(0, 16)],
                        accs.at[pl.ds(sid * 16, 16)])

    plsc.subcore_barrier()

    wid = cid * NS + sid
    ebase0 = wid * EPT

    def block(bi, carry):
        eb = ebase0 + bi * G
        pltpu.sync_copy(src_hbm.at[pl.ds(eb, G)], srcb)
        pltpu.sync_copy(dst_hbm.at[pl.ds(eb, G)], dstb)
        pltpu.sync_copy(ew_hbm.at[pl.ds(eb, G)], ewb)
        cp1 = pltpu.async_copy(qe_hbm.at[dstb], qrows, sem1)
        cp2 = pltpu.async_copy(k_hbm.at[srcb], krows, sem2)
        cp3 = pltpu.async_copy(v_hbm.at[srcb], vrows, sem3)
        cp1.wait()
        cp2.wait()
        cp3.wait()
        for g in range(G // LN):
            jv = lanes + (g * LN)
            dv16 = dstb[pl.ds(g * LN, LN)]

            def dotstep(dd, acc16):
                dv = jnp.zeros((LN,), jnp.int32) + dd
                qv = plsc.load_gather(qrows, [jv, dv])
                kv = plsc.load_gather(krows, [jv, dv])
                return acc16 + qv * kv

            dot = lax.fori_loop(0, H, dotstep, jnp.zeros((LN,), jnp.float32),
                                unroll=4)
            tv = plsc.load_gather(qrows, [jv, jnp.full((LN,), H, jnp.int32)])
            ewv = ewb[pl.ds(g * LN, LN)]
            ex = jnp.exp(dot + ewv * tv)
            exw = ex * ewv
            srcb[pl.ds(g * LN, LN)] = dv16 & 127
            for j in range(LN):
                jj = g * LN + j
                exj = lax.index_in_dim(ex, j, keepdims=False)
                exwj = lax.index_in_dim(exw, j, keepdims=False)
                for r in range(8):
                    vrows[jj, pl.ds(r * LN, LN)] = (
                        vrows[jj, pl.ds(r * LN, LN)] * exj
                        + wecb[pl.ds(r * LN, LN)] * exwj)
                for r in range(8):
                    krows[jj, pl.ds(r * LN, LN)] = zvec
            plsc.store_scatter(krows, [jv, dv16 >> 7], ex)
        pltpu.sync_copy(vrows, accv.at[dstb], add=True)
        pltpu.sync_copy(krows, accs.at[srcb], add=True)
        return carry

    lax.fori_loop(0, NBLK, block, 0)
    plsc.subcore_barrier()

    @pl.when(sid < NS - 1)
    def _():
        pltpu.sync_copy(accv.at[pl.ds(sid * RPT, RPT)],
                        outv_hbm.at[cid, pl.ds(sid * RPT, RPT)])

    @pl.when(sid == NS - 1)
    def _():
        pltpu.sync_copy(accv.at[pl.ds((NS - 1) * RPT, N - (NS - 1) * RPT)],
                        outv_hbm.at[cid, pl.ds((NS - 1) * RPT,
                                               N - (NS - 1) * RPT)])

    @pl.when(sid < NS_ROWS // 16)
    def _():
        pltpu.sync_copy(accs.at[pl.ds(sid * 16, 16)],
                        outs_hbm.at[cid, pl.ds(sid * 16, 16)])


_edge_kernel = pl.kernel(_edge_body, **_EDGE_KW)


def _layer_weights(Wq, bq, Wk, bk, Wv, bv, We, Ws, bs):
    rs = 1.0 / jnp.sqrt(float(H))
    wec = We[:, 0]
    wt = (Wq.T @ wec) * rs
    bt = jnp.dot(bq, wec) * rs
    W = jnp.concatenate([
        Wq.T * rs, wt[:, None], jnp.zeros((H, 127), jnp.float32),
        Wk.T, Wv.T, Ws.T], axis=1)
    b = jnp.concatenate([
        bq * rs, bt[None], jnp.zeros((127,), jnp.float32),
        bk, bv, bs])
    return W, b[None, :], wec


def kernel(x, edge_index, edge_weight, batch,
           Wq1, bq1, Wk1, bk1, Wv1, bv1, We1, Ws1, bs1,
           Wq2, bq2, Wk2, bk2, Wv2, bv2, We2, Ws2, bs2,
           Wl, bl):
    src = edge_index[0]
    dst = edge_index[1]
    ew = edge_weight

    W1, b1, wec1 = _layer_weights(Wq1, bq1, Wk1, bk1, Wv1, bv1, We1, Ws1, bs1)
    W2, b2, wec2 = _layer_weights(Wq2, bq2, Wk2, bk2, Wv2, bv2, We2, Ws2, bs2)
    zero = jnp.zeros((RPT, 128), jnp.float32)

    xp = jnp.pad(x, ((0, NP - N), (0, 0)))
    Q1, K1, V1, S1 = _lift(xp, W1, b1)
    av1, as1 = _edge_kernel(Q1, K1, V1, src, dst, ew, wec1, zero)
    Q2, K2, V2, S2 = _mid(av1[0], av1[1], as1[0], as1[1], S1, W2, b2)
    av2, as2 = _edge_kernel(Q2, K2, V2, src, dst, ew, wec2, zero)

    batch3d = jnp.pad(batch, (0, NP - N), constant_values=B).reshape(
        NBLOCKS, 1, 128)
    wlt = jnp.zeros((H, H), jnp.float32).at[:, :OUT].set(Wl.T)
    blp = jnp.zeros((1, H), jnp.float32).at[0, :OUT].set(bl)
    out = _fin(av2[0], av2[1], as2[0], as2[1], S2, batch3d, wlt, blp)
    return out[:, :OUT]
